# initial kernel scaffold (unmeasured)
import jax
import jax.numpy as jnp
from jax import lax
from jax.experimental import pallas as pl
from jax.experimental.pallas import tpu as pltpu

N_DEV = 4


def kernel(x, w_mat, scale_x, scale_w):
    m_per, k = x.shape
    n_per = w_mat.shape[1] // N_DEV

    my = lax.axis_index("i")
    w_cols = lax.dynamic_slice_in_dim(w_mat, my * n_per, n_per, axis=1)
    w8 = w_cols.astype(jnp.float8_e4m3fn)
    x8 = x.astype(jnp.float8_e4m3fn)

    def body(x_ref, w_ref, sx_ref, sw_ref, out_ref, comm_ref, send_sems, recv_sems):
        my_pos = lax.axis_index("i")
        left = lax.rem(my_pos + (N_DEV - 1), N_DEV)
        right = lax.rem(my_pos + 1, N_DEV)

        barrier_sem = pltpu.get_barrier_semaphore()
        for nbr in (left, right):
            pl.semaphore_signal(
                barrier_sem, inc=1,
                device_id=(nbr,), device_id_type=pl.DeviceIdType.MESH,
            )
        pl.semaphore_wait(barrier_sem, 2)

        scale = sx_ref[0] * sw_ref[0]

        def compute(chunk, origin):
            acc = lax.dot_general(
                chunk, w_ref[...],
                (((1,), (0,)), ((), ())),
                preferred_element_type=jnp.float32,
            )
            y = acc * scale
            out_ref[pl.ds(origin * m_per, m_per), :] = y * jax.nn.sigmoid(y)

        comm_ref[0] = x_ref[...]
        compute(x_ref[...], my_pos)

        for h in range(N_DEV - 1):
            s, r = h % 2, (h + 1) % 2
            rdma = pltpu.make_async_remote_copy(
                src_ref=comm_ref.at[s],
                dst_ref=comm_ref.at[r],
                send_sem=send_sems.at[s],
                recv_sem=recv_sems.at[r],
                device_id=(right,),
                device_id_type=pl.DeviceIdType.MESH,
            )
            rdma.start()
            rdma.wait()
            origin = lax.rem(my_pos + (N_DEV - 1 - h), N_DEV)
            compute(comm_ref[r], origin)

    return pl.pallas_call(
        body,
        out_shape=jax.ShapeDtypeStruct((N_DEV * m_per, n_per), jnp.float32),
        in_specs=[
            pl.BlockSpec(memory_space=pltpu.VMEM),
            pl.BlockSpec(memory_space=pltpu.VMEM),
            pl.BlockSpec(memory_space=pltpu.SMEM),
            pl.BlockSpec(memory_space=pltpu.SMEM),
        ],
        out_specs=pl.BlockSpec(memory_space=pltpu.VMEM),
        scratch_shapes=[
            pltpu.VMEM((2, m_per, k), jnp.float8_e4m3fn),
            pltpu.SemaphoreType.DMA((2,)),
            pltpu.SemaphoreType.DMA((2,)),
        ],
        compiler_params=pltpu.CompilerParams(collective_id=0),
    )(x8, w8, scale_x, scale_w)


# baseline (device time: 251568 ns/iter reference)
import jax
import jax.numpy as jnp
from jax import lax
from jax.experimental import pallas as pl
from jax.experimental.pallas import tpu as pltpu

N_DEV = 4


def kernel(x, w_mat, scale_x, scale_w):
    m_per, k = x.shape
    n_per = w_mat.shape[1] // N_DEV

    my = lax.axis_index("i")
    w_cols = lax.dynamic_slice_in_dim(w_mat, my * n_per, n_per, axis=1)
    w8 = w_cols.astype(jnp.float8_e4m3fn)
    x8 = x.astype(jnp.float8_e4m3fn)

    def body(x_ref, w_ref, sx_ref, sw_ref, out_ref, comm_ref, send_sems, recv_sems):
        my_pos = lax.axis_index("i")
        left = lax.rem(my_pos + (N_DEV - 1), N_DEV)
        right = lax.rem(my_pos + 1, N_DEV)

        barrier_sem = pltpu.get_barrier_semaphore()
        for nbr in (left, right):
            pl.semaphore_signal(
                barrier_sem, inc=1,
                device_id=(nbr,), device_id_type=pl.DeviceIdType.MESH,
            )
        pl.semaphore_wait(barrier_sem, 2)

        scale = sx_ref[0] * sw_ref[0]

        def compute(chunk, origin):
            acc = lax.dot_general(
                chunk, w_ref[...],
                (((1,), (0,)), ((), ())),
                preferred_element_type=jnp.float32,
            )
            y = acc * scale
            out_ref[pl.ds(origin * m_per, m_per), :] = y * jax.nn.sigmoid(y)

        comm_ref[0] = x_ref[...]
        compute(x_ref[...], my_pos)

        for h in range(N_DEV - 1):
            s, r = h % 2, (h + 1) % 2
            rdma = pltpu.make_async_remote_copy(
                src_ref=comm_ref.at[s],
                dst_ref=comm_ref.at[r],
                send_sem=send_sems.at[s],
                recv_sem=recv_sems.at[r],
                device_id=(right,),
                device_id_type=pl.DeviceIdType.MESH,
            )
            rdma.start()
            rdma.wait()
            origin = lax.rem(my_pos + (N_DEV - 1 - h), N_DEV)
            compute(comm_ref[r], origin)

    return pl.pallas_call(
        body,
        out_shape=jax.ShapeDtypeStruct((N_DEV * m_per, n_per), jnp.float32),
        in_specs=[
            pl.BlockSpec(memory_space=pltpu.VMEM),
            pl.BlockSpec(memory_space=pltpu.VMEM),
            pl.BlockSpec(memory_space=pltpu.SMEM),
            pl.BlockSpec(memory_space=pltpu.SMEM),
        ],
        out_specs=pl.BlockSpec(memory_space=pltpu.VMEM),
        scratch_shapes=[
            pltpu.VMEM((2, m_per, k), jnp.float8_e4m3fn),
            pltpu.SemaphoreType.DMA((2,)),
            pltpu.SemaphoreType.DMA((2,)),
        ],
        compiler_params=pltpu.CompilerParams(
            collective_id=0,
            vmem_limit_bytes=100 * 1024 * 1024,
        ),
    )(x8, w8, scale_x, scale_w)


# device time: 151828 ns/iter; 1.6569x vs baseline; 1.6569x over previous
import jax
import jax.numpy as jnp
from jax import lax
from jax.experimental import pallas as pl
from jax.experimental.pallas import tpu as pltpu

N_DEV = 4


def kernel(x, w_mat, scale_x, scale_w):
    m_per, k = x.shape
    n_per = w_mat.shape[1] // N_DEV
    m_half = m_per // 2

    my = lax.axis_index("i")
    w_cols = lax.dynamic_slice_in_dim(w_mat, my * n_per, n_per, axis=1)
    w8 = w_cols.astype(jnp.float8_e4m3fn)
    x8 = x.astype(jnp.float8_e4m3fn)

    def body(x_ref, w_ref, sx_ref, sw_ref, out_ref,
             cw_ref, ccw_ref, cw_send, cw_recv, ccw_send, ccw_recv):
        my_pos = lax.axis_index("i")
        left = lax.rem(my_pos + (N_DEV - 1), N_DEV)
        right = lax.rem(my_pos + 1, N_DEV)

        barrier_sem = pltpu.get_barrier_semaphore()
        for nbr in (left, right):
            pl.semaphore_signal(
                barrier_sem, inc=1,
                device_id=(nbr,), device_id_type=pl.DeviceIdType.MESH,
            )
        pl.semaphore_wait(barrier_sem, 2)

        scale = sx_ref[0] * sw_ref[0]

        def compute(chunk, row_start):
            acc = lax.dot_general(
                chunk, w_ref[...],
                (((1,), (0,)), ((), ())),
                preferred_element_type=jnp.float32,
            )
            y = acc * scale
            out_ref[pl.ds(row_start, chunk.shape[0]), :] = y * jax.nn.sigmoid(y)

        cw_ref[0] = x_ref[:m_half, :]
        ccw_ref[0] = x_ref[m_half:, :]

        def hop(comm_ref, send_sems, recv_sems, h, target):
            return pltpu.make_async_remote_copy(
                src_ref=comm_ref.at[h],
                dst_ref=comm_ref.at[h + 1],
                send_sem=send_sems.at[h],
                recv_sem=recv_sems.at[h],
                device_id=(target,),
                device_id_type=pl.DeviceIdType.MESH,
            )

        cw = [hop(cw_ref, cw_send, cw_recv, h, right) for h in range(N_DEV - 1)]
        ccw = [hop(ccw_ref, ccw_send, ccw_recv, h, left) for h in range(N_DEV - 1)]
        cw[0].start()
        ccw[0].start()

        compute(x_ref[...], my_pos * m_per)

        for h in range(N_DEV - 1):
            cw[h].wait_recv()
            if h + 1 < N_DEV - 1:
                cw[h + 1].start()
            ccw[h].wait_recv()
            if h + 1 < N_DEV - 1:
                ccw[h + 1].start()

            origin_cw = lax.rem(my_pos + (N_DEV - 1 - h), N_DEV)
            origin_ccw = lax.rem(my_pos + h + 1, N_DEV)
            compute(cw_ref[h + 1], origin_cw * m_per)
            compute(ccw_ref[h + 1], origin_ccw * m_per + m_half)

        for h in range(N_DEV - 1):
            cw[h].wait_send()
            ccw[h].wait_send()

    return pl.pallas_call(
        body,
        out_shape=jax.ShapeDtypeStruct((N_DEV * m_per, n_per), jnp.float32),
        in_specs=[
            pl.BlockSpec(memory_space=pltpu.VMEM),
            pl.BlockSpec(memory_space=pltpu.VMEM),
            pl.BlockSpec(memory_space=pltpu.SMEM),
            pl.BlockSpec(memory_space=pltpu.SMEM),
        ],
        out_specs=pl.BlockSpec(memory_space=pltpu.VMEM),
        scratch_shapes=[
            pltpu.VMEM((N_DEV, m_half, k), jnp.float8_e4m3fn),
            pltpu.VMEM((N_DEV, m_half, k), jnp.float8_e4m3fn),
            pltpu.SemaphoreType.DMA((N_DEV - 1,)),
            pltpu.SemaphoreType.DMA((N_DEV - 1,)),
            pltpu.SemaphoreType.DMA((N_DEV - 1,)),
            pltpu.SemaphoreType.DMA((N_DEV - 1,)),
        ],
        compiler_params=pltpu.CompilerParams(
            collective_id=0,
            vmem_limit_bytes=100 * 1024 * 1024,
        ),
    )(x8, w8, scale_x, scale_w)


# device time: 117434 ns/iter; 2.1422x vs baseline; 1.2929x over previous
import jax
import jax.numpy as jnp
from jax import lax
from jax.experimental import pallas as pl
from jax.experimental.pallas import tpu as pltpu

N_DEV = 4
K_SLICES = 8


def kernel(x, w_mat, scale_x, scale_w):
    m_per, k = x.shape
    n_per = w_mat.shape[1] // N_DEV
    m_half = m_per // 2
    k_step = k // K_SLICES

    def body(x_ref, w_hbm, sx_ref, sw_ref, out_hbm,
             cw_ref, ccw_ref, wstage_ref, w8_ref, ostage_ref,
             cw_send, cw_recv, ccw_send, ccw_recv, w_sems, o_sems):
        my_pos = lax.axis_index("i")
        left = lax.rem(my_pos + (N_DEV - 1), N_DEV)
        right = lax.rem(my_pos + 1, N_DEV)

        cw_ref[0] = x_ref[:m_half, :].astype(jnp.float8_e4m3fn)
        ccw_ref[0] = x_ref[m_half:, :].astype(jnp.float8_e4m3fn)

        barrier_sem = pltpu.get_barrier_semaphore()
        for nbr in (left, right):
            pl.semaphore_signal(
                barrier_sem, inc=1,
                device_id=(nbr,), device_id_type=pl.DeviceIdType.MESH,
            )
        pl.semaphore_wait(barrier_sem, 2)

        def hop(comm_ref, send_sems, recv_sems, h, target):
            return pltpu.make_async_remote_copy(
                src_ref=comm_ref.at[h],
                dst_ref=comm_ref.at[h + 1],
                send_sem=send_sems.at[h],
                recv_sem=recv_sems.at[h],
                device_id=(target,),
                device_id_type=pl.DeviceIdType.MESH,
            )

        cw = [hop(cw_ref, cw_send, cw_recv, h, right) for h in range(N_DEV - 1)]
        ccw = [hop(ccw_ref, ccw_send, ccw_recv, h, left) for h in range(N_DEV - 1)]
        cw[0].start()
        ccw[0].start()

        def w_fetch(s):
            return pltpu.make_async_copy(
                w_hbm.at[pl.ds(s * k_step, k_step), pl.ds(my_pos * n_per, n_per)],
                wstage_ref.at[s % 2],
                w_sems.at[s % 2],
            )

        w_fetch(0).start()
        w_fetch(1).start()
        for s in range(K_SLICES):
            w_fetch(s).wait()
            w8_ref[pl.ds(s * k_step, k_step), :] = (
                wstage_ref[s % 2].astype(jnp.float8_e4m3fn))
            if s + 2 < K_SLICES:
                w_fetch(s + 2).start()

        scale = sx_ref[0] * sw_ref[0]

        out_copies = [None] * (2 * N_DEV)

        def compute(chunk, row_start, idx):
            buf = idx % 2
            if idx >= 2:
                out_copies[idx - 2].wait()
            acc = lax.dot_general(
                chunk, w8_ref[...],
                (((1,), (0,)), ((), ())),
                preferred_element_type=jnp.float32,
            )
            y = acc * scale
            ostage_ref[buf] = y * jax.nn.sigmoid(y)
            cp = pltpu.make_async_copy(
                ostage_ref.at[buf],
                out_hbm.at[pl.ds(row_start, m_half), :],
                o_sems.at[buf],
            )
            cp.start()
            out_copies[idx] = cp

        compute(cw_ref[0], my_pos * m_per, 0)
        compute(ccw_ref[0], my_pos * m_per + m_half, 1)

        for h in range(N_DEV - 1):
            cw[h].wait_recv()
            if h + 1 < N_DEV - 1:
                cw[h + 1].start()
            ccw[h].wait_recv()
            if h + 1 < N_DEV - 1:
                ccw[h + 1].start()

            origin_cw = lax.rem(my_pos + (N_DEV - 1 - h), N_DEV)
            origin_ccw = lax.rem(my_pos + h + 1, N_DEV)
            compute(cw_ref[h + 1], origin_cw * m_per, 2 * h + 2)
            compute(ccw_ref[h + 1], origin_ccw * m_per + m_half, 2 * h + 3)

        out_copies[2 * N_DEV - 2].wait()
        out_copies[2 * N_DEV - 1].wait()
        for h in range(N_DEV - 1):
            cw[h].wait_send()
            ccw[h].wait_send()

    return pl.pallas_call(
        body,
        out_shape=jax.ShapeDtypeStruct((N_DEV * m_per, n_per), jnp.float32),
        in_specs=[
            pl.BlockSpec(memory_space=pltpu.VMEM),
            pl.BlockSpec(memory_space=pltpu.MemorySpace.HBM),
            pl.BlockSpec(memory_space=pltpu.SMEM),
            pl.BlockSpec(memory_space=pltpu.SMEM),
        ],
        out_specs=pl.BlockSpec(memory_space=pltpu.MemorySpace.HBM),
        scratch_shapes=[
            pltpu.VMEM((N_DEV, m_half, k), jnp.float8_e4m3fn),
            pltpu.VMEM((N_DEV, m_half, k), jnp.float8_e4m3fn),
            pltpu.VMEM((2, k // K_SLICES, n_per), jnp.float32),
            pltpu.VMEM((k, n_per), jnp.float8_e4m3fn),
            pltpu.VMEM((2, m_half, n_per), jnp.float32),
            pltpu.SemaphoreType.DMA((N_DEV - 1,)),
            pltpu.SemaphoreType.DMA((N_DEV - 1,)),
            pltpu.SemaphoreType.DMA((N_DEV - 1,)),
            pltpu.SemaphoreType.DMA((N_DEV - 1,)),
            pltpu.SemaphoreType.DMA((2,)),
            pltpu.SemaphoreType.DMA((2,)),
        ],
        compiler_params=pltpu.CompilerParams(
            collective_id=0,
            vmem_limit_bytes=100 * 1024 * 1024,
        ),
    )(x, w_mat, scale_x, scale_w)


# device time: 108285 ns/iter; 2.3232x vs baseline; 1.0845x over previous
import jax
import jax.numpy as jnp
from jax import lax
from jax.experimental import pallas as pl
from jax.experimental.pallas import tpu as pltpu

N_DEV = 4
SUBS = 2
K_SLICES = 8
OBUF = 4

def kernel(x, w_mat, scale_x, scale_w):
    m_per, k = x.shape
    n_per = w_mat.shape[1] // N_DEV
    m_half = m_per // 2
    m_sub = m_half // SUBS
    k_step = k // K_SLICES

    def body(x_ref, w_hbm, sx_ref, sw_ref, out_hbm,
             cw_ref, ccw_ref, wstage_ref, w8_ref, ostage_ref,
             cw_send, cw_recv, ccw_send, ccw_recv, w_sems, o_sems):
        my_pos = lax.axis_index("i")
        left = lax.rem(my_pos + (N_DEV - 1), N_DEV)
        right = lax.rem(my_pos + 1, N_DEV)

        cw_ref[0] = x_ref[:m_half, :].astype(jnp.float8_e4m3fn)
        ccw_ref[0] = x_ref[m_half:, :].astype(jnp.float8_e4m3fn)

        barrier_sem = pltpu.get_barrier_semaphore()
        for nbr in (left, right):
            pl.semaphore_signal(
                barrier_sem, inc=1,
                device_id=(nbr,), device_id_type=pl.DeviceIdType.MESH,
            )
        pl.semaphore_wait(barrier_sem, 2)

        def hop(comm_ref, send_sems, recv_sems, h, j, target):
            return pltpu.make_async_remote_copy(
                src_ref=comm_ref.at[h, pl.ds(j * m_sub, m_sub)],
                dst_ref=comm_ref.at[h + 1, pl.ds(j * m_sub, m_sub)],
                send_sem=send_sems.at[h, j],
                recv_sem=recv_sems.at[h, j],
                device_id=(target,),
                device_id_type=pl.DeviceIdType.MESH,
            )

        cw = [[hop(cw_ref, cw_send, cw_recv, h, j, right) for j in range(SUBS)]
              for h in range(N_DEV - 1)]
        ccw = [[hop(ccw_ref, ccw_send, ccw_recv, h, j, left) for j in range(SUBS)]
               for h in range(N_DEV - 1)]
        for j in range(SUBS):
            cw[0][j].start()
            ccw[0][j].start()

        def w_fetch(s):
            return pltpu.make_async_copy(
                w_hbm.at[pl.ds(s * k_step, k_step), pl.ds(my_pos * n_per, n_per)],
                wstage_ref.at[s % 2],
                w_sems.at[s % 2],
            )

        w_fetch(0).start()
        w_fetch(1).start()
        for s in range(K_SLICES):
            w_fetch(s).wait()
            w8_ref[pl.ds(s * k_step, k_step), :] = (
                wstage_ref[s % 2].astype(jnp.float8_e4m3fn))
            if s + 2 < K_SLICES:
                w_fetch(s + 2).start()

        scale = sx_ref[0] * sw_ref[0]

        n_blocks = 2 * SUBS * N_DEV
        out_copies = [None] * n_blocks

        def compute(chunk, row_start, idx):
            buf = idx % OBUF
            acc = lax.dot_general(
                chunk, w8_ref[...],
                (((1,), (0,)), ((), ())),
                preferred_element_type=jnp.float32,
            )
            if idx >= OBUF:
                out_copies[idx - OBUF].wait()
            y = acc * scale
            ostage_ref[buf] = y * jax.nn.sigmoid(y)
            cp = pltpu.make_async_copy(
                ostage_ref.at[buf],
                out_hbm.at[pl.ds(row_start, m_sub), :],
                o_sems.at[buf],
            )
            cp.start()
            out_copies[idx] = cp

        idx = 0
        for j in range(SUBS):
            compute(cw_ref[0, pl.ds(j * m_sub, m_sub)],
                    my_pos * m_per + j * m_sub, idx); idx += 1
            compute(ccw_ref[0, pl.ds(j * m_sub, m_sub)],
                    my_pos * m_per + m_half + j * m_sub, idx); idx += 1

        for h in range(N_DEV - 1):
            origin_cw = lax.rem(my_pos + (N_DEV - 1 - h), N_DEV)
            origin_ccw = lax.rem(my_pos + h + 1, N_DEV)
            for j in range(SUBS):
                cw[h][j].wait_recv()
                if h + 1 < N_DEV - 1:
                    cw[h + 1][j].start()
                ccw[h][j].wait_recv()
                if h + 1 < N_DEV - 1:
                    ccw[h + 1][j].start()
                compute(cw_ref[h + 1, pl.ds(j * m_sub, m_sub)],
                        origin_cw * m_per + j * m_sub, idx); idx += 1
                compute(ccw_ref[h + 1, pl.ds(j * m_sub, m_sub)],
                        origin_ccw * m_per + m_half + j * m_sub, idx); idx += 1

        for i in range(n_blocks - OBUF, n_blocks):
            out_copies[i].wait()
        for h in range(N_DEV - 1):
            for j in range(SUBS):
                cw[h][j].wait_send()
                ccw[h][j].wait_send()

    return pl.pallas_call(
        body,
        out_shape=jax.ShapeDtypeStruct((N_DEV * m_per, n_per), jnp.float32),
        in_specs=[
            pl.BlockSpec(memory_space=pltpu.VMEM),
            pl.BlockSpec(memory_space=pltpu.MemorySpace.HBM),
            pl.BlockSpec(memory_space=pltpu.SMEM),
            pl.BlockSpec(memory_space=pltpu.SMEM),
        ],
        out_specs=pl.BlockSpec(memory_space=pltpu.MemorySpace.HBM),
        scratch_shapes=[
            pltpu.VMEM((N_DEV, m_half, k), jnp.float8_e4m3fn),
            pltpu.VMEM((N_DEV, m_half, k), jnp.float8_e4m3fn),
            pltpu.VMEM((2, k // K_SLICES, n_per), jnp.float32),
            pltpu.VMEM((k, n_per), jnp.float8_e4m3fn),
            pltpu.VMEM((OBUF, m_half // SUBS, n_per), jnp.float32),
            pltpu.SemaphoreType.DMA((N_DEV - 1, SUBS)),
            pltpu.SemaphoreType.DMA((N_DEV - 1, SUBS)),
            pltpu.SemaphoreType.DMA((N_DEV - 1, SUBS)),
            pltpu.SemaphoreType.DMA((N_DEV - 1, SUBS)),
            pltpu.SemaphoreType.DMA((2,)),
            pltpu.SemaphoreType.DMA((OBUF,)),
        ],
        compiler_params=pltpu.CompilerParams(
            collective_id=0,
            vmem_limit_bytes=100 * 1024 * 1024,
        ),
    )(x, w_mat, scale_x, scale_w)
